# baseline (speedup 1.0000x reference)
import functools

import jax
import jax.numpy as jnp
from jax import lax
from jax.experimental import pallas as pl
from jax.experimental.pallas import tpu as pltpu
from jax.experimental.pallas import tpu_sc as plsc

B = 2048
D = 768
O = 768
E = 16
BLKR = 256
NBLKMAX = (B * 2) // BLKR + E
XSROWS = NBLKMAX * BLKR
HD = D // 2
NEG_INF = float("-inf")

_NC = 2
_NS = 16
_NW = _NC * _NS
_RPW = B // _NW


def _route_kernel(x_ref, w_bid_ref, b_bid_ref,
                  bids_ref, idx_ref, pos_ref, bmap_ref, tot_ref, xb_ref):
    iota = jax.lax.broadcasted_iota(jnp.int32, (B, E), 1)
    x = x_ref[...]
    bids = jax.lax.dot_general(
        x, w_bid_ref[...], (((1,), (1,)), ((), ())),
        preferred_element_type=jnp.float32) + b_bid_ref[...]
    bids_ref[...] = bids
    max0 = jnp.max(bids, axis=1, keepdims=True)
    i0 = jnp.min(jnp.where(bids == max0, iota, E), axis=1, keepdims=True)
    masked = jnp.where(iota == i0, NEG_INF, bids)
    max1 = jnp.max(masked, axis=1, keepdims=True)
    i1 = jnp.min(jnp.where(masked == max1, iota, E), axis=1, keepdims=True)
    idx_ref[...] = jnp.concatenate([i0, i1], axis=1)
    coeff2 = jnp.where((iota == i0) | (iota == i1), 1.0, 0.0)
    s = coeff2
    sh = 1
    while sh < B:
        s = s + jnp.concatenate(
            [jnp.zeros((sh, E), jnp.float32), s[:B - sh, :]], axis=0)
        sh *= 2
    s_excl = s - coeff2
    r0 = jnp.sum(jnp.where(iota == i0, s_excl, 0.0), axis=1, keepdims=True)
    r1 = jnp.sum(jnp.where(iota == i1, s_excl, 0.0), axis=1, keepdims=True)
    totals = s[B - 1:B, :]
    nblk = jnp.floor((totals + (BLKR - 1.0)) * (1.0 / BLKR))
    sb = nblk
    sh = 1
    while sh < E:
        sb = sb + jnp.concatenate(
            [jnp.zeros((1, sh), jnp.float32), sb[:, :E - sh]], axis=1)
        sh *= 2
    start = sb - nblk
    tot = sb[:, E - 1:E]
    off = start * float(BLKR)
    o0 = jnp.sum(jnp.where(iota == i0, off, 0.0), axis=1, keepdims=True)
    o1 = jnp.sum(jnp.where(iota == i1, off, 0.0), axis=1, keepdims=True)
    pos0 = (o0 + r0).astype(jnp.int32)
    pos1 = (o1 + r1).astype(jnp.int32)
    pos_ref[...] = jnp.concatenate([pos0, pos1], axis=1)
    blk_i = jax.lax.broadcasted_iota(jnp.int32, (NBLKMAX, 1), 0
                                     ).astype(jnp.float32)
    blk_i = jnp.minimum(blk_i, tot - 1.0)
    cmp = jnp.where(start <= blk_i, 1.0, 0.0)
    bmap_ref[...] = (jnp.sum(cmp, axis=1, keepdims=True) - 1.0
                     ).astype(jnp.int32)
    tot_ref[...] = tot.astype(jnp.int32)
    xlo = jax.lax.bitcast_convert_type(
        x[:, :HD].astype(jnp.bfloat16).astype(jnp.float32), jnp.uint32)
    xhi = jax.lax.bitcast_convert_type(
        x[:, HD:].astype(jnp.bfloat16).astype(jnp.float32), jnp.uint32)
    xb_ref[...] = jax.lax.bitcast_convert_type(
        xhi | (xlo >> 16), jnp.float32)


def _route(x, W_bid, b_bid):
    return pl.pallas_call(
        _route_kernel,
        in_specs=[
            pl.BlockSpec((B, D), lambda: (0, 0)),
            pl.BlockSpec((E, D), lambda: (0, 0)),
            pl.BlockSpec((1, E), lambda: (0, 0)),
        ],
        out_specs=[
            pl.BlockSpec((B, E), lambda: (0, 0)),
            pl.BlockSpec((B, 2), lambda: (0, 0)),
            pl.BlockSpec((B, 2), lambda: (0, 0)),
            pl.BlockSpec((NBLKMAX, 1), lambda: (0, 0)),
            pl.BlockSpec((1, 1), lambda: (0, 0)),
            pl.BlockSpec((B, HD), lambda: (0, 0)),
        ],
        out_shape=[
            jax.ShapeDtypeStruct((B, E), jnp.float32),
            jax.ShapeDtypeStruct((B, 2), jnp.int32),
            jax.ShapeDtypeStruct((B, 2), jnp.int32),
            jax.ShapeDtypeStruct((NBLKMAX, 1), jnp.int32),
            jax.ShapeDtypeStruct((1, 1), jnp.int32),
            jax.ShapeDtypeStruct((B, HD), jnp.float32),
        ],
    )(x, W_bid, b_bid.reshape(1, E))


@functools.cache
def _sc_mesh():
    return plsc.VectorSubcoreMesh(core_axis_name="c", subcore_axis_name="s",
                                  num_cores=_NC, num_subcores=_NS)


@functools.cache
def _make_sc_dispatch():
    @functools.partial(
        pl.kernel,
        out_type=jax.ShapeDtypeStruct((XSROWS, HD), jnp.float32),
        mesh=_sc_mesh(),
        scratch_types=[
            pltpu.VMEM((_RPW, HD), jnp.float32),
            pltpu.VMEM((_RPW,), jnp.int32),
            pltpu.VMEM((_RPW,), jnp.int32),
            pltpu.SemaphoreType.DMA,
            pltpu.SemaphoreType.DMA,
        ],
    )
    def _sc_dispatch(x_hbm, pos0_hbm, pos1_hbm, xs_hbm,
                     rows_v, p0_v, p1_v, sem0, sem1):
        wid = lax.axis_index("s") * _NC + lax.axis_index("c")
        base = wid * _RPW
        pltpu.sync_copy(x_hbm.at[pl.ds(base, _RPW)], rows_v)
        pltpu.sync_copy(pos0_hbm.at[pl.ds(base, _RPW)], p0_v)
        pltpu.sync_copy(pos1_hbm.at[pl.ds(base, _RPW)], p1_v)
        c0 = pltpu.async_copy(rows_v, xs_hbm.at[p0_v], sem0)
        c1 = pltpu.async_copy(rows_v, xs_hbm.at[p1_v], sem1)
        c0.wait()
        c1.wait()

    return _sc_dispatch


def _mm_body(bmap_ref, tot_ref, xs_ref, wa_ref, wb_ref, bo_ref, ys_ref):
    i = pl.program_id(0)

    @pl.when(i < tot_ref[0])
    def _():
        u = jax.lax.bitcast_convert_type(xs_ref[...], jnp.uint32)
        lo = jax.lax.bitcast_convert_type(
            u << 16, jnp.float32).astype(jnp.bfloat16)
        hi = jax.lax.bitcast_convert_type(
            u & jnp.uint32(0xFFFF0000), jnp.float32).astype(jnp.bfloat16)
        dn = (((1,), (0,)), ((), ()))
        for half, w_ref in ((0, wa_ref), (1, wb_ref)):
            w = w_ref[0].astype(jnp.bfloat16)
            acc = (jax.lax.dot_general(lo, w[:HD], dn,
                                       preferred_element_type=jnp.float32)
                   + jax.lax.dot_general(hi, w[HD:], dn,
                                         preferred_element_type=jnp.float32))
            ys_ref[:, half * HD:(half + 1) * HD] = (
                0.5 * acc + 0.5 * bo_ref[0][:, half * HD:(half + 1) * HD])


def _grouped_mm(bmap, tot, xs, W_out, b_out):
    grid_spec = pltpu.PrefetchScalarGridSpec(
        num_scalar_prefetch=2,
        grid=(NBLKMAX,),
        in_specs=[
            pl.BlockSpec((BLKR, HD),
                         lambda i, bm, tt: (jnp.minimum(i, tt[0] - 1), 0)),
            pl.BlockSpec((1, D, HD), lambda i, bm, tt: (bm[i], 0, 0)),
            pl.BlockSpec((1, D, HD), lambda i, bm, tt: (bm[i], 0, 1)),
            pl.BlockSpec((1, 1, O), lambda i, bm, tt: (bm[i], 0, 0)),
        ],
        out_specs=pl.BlockSpec((BLKR, O),
                               lambda i, bm, tt: (jnp.minimum(i, tt[0] - 1), 0)),
    )
    return pl.pallas_call(
        _mm_body,
        grid_spec=grid_spec,
        out_shape=jax.ShapeDtypeStruct((XSROWS, O), jnp.float32),
    )(bmap, tot, xs, W_out, W_out, b_out.reshape(E, 1, O))


@functools.cache
def _make_sc_combine():
    @functools.partial(
        pl.kernel,
        out_type=jax.ShapeDtypeStruct((B, O), jnp.float32),
        mesh=_sc_mesh(),
        scratch_types=[
            pltpu.VMEM((_RPW, O), jnp.float32),
            pltpu.VMEM((_RPW, O), jnp.float32),
            pltpu.VMEM((_RPW,), jnp.int32),
            pltpu.VMEM((_RPW,), jnp.int32),
            pltpu.SemaphoreType.DMA,
            pltpu.SemaphoreType.DMA,
        ],
    )
    def _sc_combine(ys_hbm, pos0_hbm, pos1_hbm, final_hbm,
                    acc_v, g1_v, p0_v, p1_v, sem0, sem1):
        wid = lax.axis_index("s") * _NC + lax.axis_index("c")
        base = wid * _RPW
        pltpu.sync_copy(pos0_hbm.at[pl.ds(base, _RPW)], p0_v)
        pltpu.sync_copy(pos1_hbm.at[pl.ds(base, _RPW)], p1_v)
        c0 = pltpu.async_copy(ys_hbm.at[p0_v], acc_v, sem0)
        c1 = pltpu.async_copy(ys_hbm.at[p1_v], g1_v, sem1)
        c0.wait()
        c1.wait()

        @plsc.parallel_loop(0, _RPW, 1, unroll=4)
        def _row(i):
            for c in range(O // 16):
                sl = (i, pl.ds(c * 16, 16))
                acc_v[sl] = acc_v[sl] + g1_v[sl]

        pltpu.sync_copy(acc_v, final_hbm.at[pl.ds(base, _RPW)])

    return _sc_combine


def kernel(x, W_out, b_out, W_bid, b_bid):
    bids, idx, pos, bmap, tot, xb = _route(x, W_bid, b_bid)
    pos0, pos1 = pos[:, 0], pos[:, 1]
    xs = _make_sc_dispatch()(xb, pos0, pos1)
    ys = _grouped_mm(bmap.reshape(NBLKMAX), tot.reshape(1), xs, W_out, b_out)
    final = _make_sc_combine()(ys, pos0, pos1)
    return final, idx, bids


# speedup vs baseline: 1.0125x; 1.0125x over previous
import functools

import jax
import jax.numpy as jnp
from jax import lax
from jax.experimental import pallas as pl
from jax.experimental.pallas import tpu as pltpu
from jax.experimental.pallas import tpu_sc as plsc

B = 2048
D = 768
O = 768
E = 16
BLKR = 256
NBLKMAX = (B * 2) // BLKR + E
XSROWS = NBLKMAX * BLKR
HD = D // 2
NEG_INF = float("-inf")

_NC = 2
_NS = 16
_NW = _NC * _NS
_RPW = B // _NW


def _route_kernel(x_ref, w_bid_ref, b_bid_ref,
                  bids_ref, idx_ref, pos_ref, bmap_ref, tot_ref, xb_ref):
    iota = jax.lax.broadcasted_iota(jnp.int32, (B, E), 1)
    x = x_ref[...]
    bids = jax.lax.dot_general(
        x, w_bid_ref[...], (((1,), (1,)), ((), ())),
        preferred_element_type=jnp.float32) + b_bid_ref[...]
    bids_ref[...] = bids
    max0 = jnp.max(bids, axis=1, keepdims=True)
    i0 = jnp.min(jnp.where(bids == max0, iota, E), axis=1, keepdims=True)
    masked = jnp.where(iota == i0, NEG_INF, bids)
    max1 = jnp.max(masked, axis=1, keepdims=True)
    i1 = jnp.min(jnp.where(masked == max1, iota, E), axis=1, keepdims=True)
    idx_ref[...] = jnp.concatenate([i0, i1], axis=1)
    coeff2 = jnp.where((iota == i0) | (iota == i1), 1.0, 0.0)
    s = coeff2
    sh = 1
    while sh < B:
        s = s + jnp.concatenate(
            [jnp.zeros((sh, E), jnp.float32), s[:B - sh, :]], axis=0)
        sh *= 2
    s_excl = s - coeff2
    r0 = jnp.sum(jnp.where(iota == i0, s_excl, 0.0), axis=1, keepdims=True)
    r1 = jnp.sum(jnp.where(iota == i1, s_excl, 0.0), axis=1, keepdims=True)
    totals = s[B - 1:B, :]
    nblk = jnp.floor((totals + (BLKR - 1.0)) * (1.0 / BLKR))
    sb = nblk
    sh = 1
    while sh < E:
        sb = sb + jnp.concatenate(
            [jnp.zeros((1, sh), jnp.float32), sb[:, :E - sh]], axis=1)
        sh *= 2
    start = sb - nblk
    tot = sb[:, E - 1:E]
    off = start * float(BLKR)
    o0 = jnp.sum(jnp.where(iota == i0, off, 0.0), axis=1, keepdims=True)
    o1 = jnp.sum(jnp.where(iota == i1, off, 0.0), axis=1, keepdims=True)
    pos0 = (o0 + r0).astype(jnp.int32)
    pos1 = (o1 + r1).astype(jnp.int32)
    pos_ref[...] = jnp.concatenate([pos0, pos1], axis=1)
    blk_i = jax.lax.broadcasted_iota(jnp.int32, (NBLKMAX, 1), 0
                                     ).astype(jnp.float32)
    blk_i = jnp.minimum(blk_i, tot - 1.0)
    cmp = jnp.where(start <= blk_i, 1.0, 0.0)
    bmap_ref[...] = (jnp.sum(cmp, axis=1, keepdims=True) - 1.0
                     ).astype(jnp.int32)
    tot_ref[...] = tot.astype(jnp.int32)
    xlo = jax.lax.bitcast_convert_type(
        x[:, :HD].astype(jnp.bfloat16).astype(jnp.float32), jnp.uint32)
    xhi = jax.lax.bitcast_convert_type(
        x[:, HD:].astype(jnp.bfloat16).astype(jnp.float32), jnp.uint32)
    xb_ref[...] = jax.lax.bitcast_convert_type(
        xhi | (xlo >> 16), jnp.float32)


def _route(x, W_bid, b_bid):
    return pl.pallas_call(
        _route_kernel,
        in_specs=[
            pl.BlockSpec((B, D), lambda: (0, 0)),
            pl.BlockSpec((E, D), lambda: (0, 0)),
            pl.BlockSpec((1, E), lambda: (0, 0)),
        ],
        out_specs=[
            pl.BlockSpec((B, E), lambda: (0, 0)),
            pl.BlockSpec((B, 2), lambda: (0, 0)),
            pl.BlockSpec((B, 2), lambda: (0, 0)),
            pl.BlockSpec((NBLKMAX, 1), lambda: (0, 0)),
            pl.BlockSpec((1, 1), lambda: (0, 0)),
            pl.BlockSpec((B, HD), lambda: (0, 0)),
        ],
        out_shape=[
            jax.ShapeDtypeStruct((B, E), jnp.float32),
            jax.ShapeDtypeStruct((B, 2), jnp.int32),
            jax.ShapeDtypeStruct((B, 2), jnp.int32),
            jax.ShapeDtypeStruct((NBLKMAX, 1), jnp.int32),
            jax.ShapeDtypeStruct((1, 1), jnp.int32),
            jax.ShapeDtypeStruct((B, HD), jnp.float32),
        ],
    )(x, W_bid, b_bid.reshape(1, E))


@functools.cache
def _sc_mesh():
    return plsc.VectorSubcoreMesh(core_axis_name="c", subcore_axis_name="s",
                                  num_cores=_NC, num_subcores=_NS)


@functools.cache
def _make_sc_dispatch():
    @functools.partial(
        pl.kernel,
        out_type=jax.ShapeDtypeStruct((XSROWS, HD), jnp.float32),
        mesh=_sc_mesh(),
        scratch_types=[
            pltpu.VMEM((_RPW, HD), jnp.float32),
            pltpu.VMEM((_RPW,), jnp.int32),
            pltpu.VMEM((_RPW,), jnp.int32),
            pltpu.SemaphoreType.DMA,
            pltpu.SemaphoreType.DMA,
        ],
    )
    def _sc_dispatch(x_hbm, pos0_hbm, pos1_hbm, xs_hbm,
                     rows_v, p0_v, p1_v, sem0, sem1):
        wid = lax.axis_index("s") * _NC + lax.axis_index("c")
        base = wid * _RPW
        pltpu.sync_copy(x_hbm.at[pl.ds(base, _RPW)], rows_v)
        pltpu.sync_copy(pos0_hbm.at[pl.ds(base, _RPW)], p0_v)
        pltpu.sync_copy(pos1_hbm.at[pl.ds(base, _RPW)], p1_v)
        c0 = pltpu.async_copy(rows_v, xs_hbm.at[p0_v], sem0)
        c1 = pltpu.async_copy(rows_v, xs_hbm.at[p1_v], sem1)
        c0.wait()
        c1.wait()

    return _sc_dispatch


def _mm_body(bmap_ref, tot_ref, xs_ref, w_ref, bo_ref, ys_ref):
    i = pl.program_id(0)

    @pl.when(i < tot_ref[0])
    def _():
        u = jax.lax.bitcast_convert_type(xs_ref[...], jnp.uint32)
        lo = jax.lax.bitcast_convert_type(
            u << 16, jnp.float32).astype(jnp.bfloat16)
        hi = jax.lax.bitcast_convert_type(
            u & jnp.uint32(0xFFFF0000), jnp.float32).astype(jnp.bfloat16)
        w = w_ref[0].astype(jnp.bfloat16)
        dn = (((1,), (0,)), ((), ()))
        acc = (jax.lax.dot_general(lo, w[:HD], dn,
                                   preferred_element_type=jnp.float32)
               + jax.lax.dot_general(hi, w[HD:], dn,
                                     preferred_element_type=jnp.float32))
        ys_ref[...] = 0.5 * acc + 0.5 * bo_ref[0]


def _grouped_mm(bmap, tot, xs, W_out, b_out):
    grid_spec = pltpu.PrefetchScalarGridSpec(
        num_scalar_prefetch=2,
        grid=(NBLKMAX,),
        in_specs=[
            pl.BlockSpec((BLKR, HD),
                         lambda i, bm, tt: (jnp.minimum(i, tt[0] - 1), 0)),
            pl.BlockSpec((1, D, O), lambda i, bm, tt: (bm[i], 0, 0)),
            pl.BlockSpec((1, 1, O), lambda i, bm, tt: (bm[i], 0, 0)),
        ],
        out_specs=pl.BlockSpec((BLKR, O),
                               lambda i, bm, tt: (jnp.minimum(i, tt[0] - 1), 0)),
    )
    return pl.pallas_call(
        _mm_body,
        grid_spec=grid_spec,
        out_shape=jax.ShapeDtypeStruct((XSROWS, O), jnp.float32),
    )(bmap, tot, xs, W_out, b_out.reshape(E, 1, O))


@functools.cache
def _make_sc_combine():
    @functools.partial(
        pl.kernel,
        out_type=jax.ShapeDtypeStruct((B, O), jnp.float32),
        mesh=_sc_mesh(),
        scratch_types=[
            pltpu.VMEM((_RPW, O), jnp.float32),
            pltpu.VMEM((_RPW, O), jnp.float32),
            pltpu.VMEM((_RPW,), jnp.int32),
            pltpu.VMEM((_RPW,), jnp.int32),
            pltpu.SemaphoreType.DMA,
            pltpu.SemaphoreType.DMA,
        ],
    )
    def _sc_combine(ys_hbm, pos0_hbm, pos1_hbm, final_hbm,
                    acc_v, g1_v, p0_v, p1_v, sem0, sem1):
        wid = lax.axis_index("s") * _NC + lax.axis_index("c")
        base = wid * _RPW
        pltpu.sync_copy(pos0_hbm.at[pl.ds(base, _RPW)], p0_v)
        pltpu.sync_copy(pos1_hbm.at[pl.ds(base, _RPW)], p1_v)
        c0 = pltpu.async_copy(ys_hbm.at[p0_v], acc_v, sem0)
        c1 = pltpu.async_copy(ys_hbm.at[p1_v], g1_v, sem1)
        c0.wait()
        c1.wait()

        @plsc.parallel_loop(0, _RPW, 1, unroll=4)
        def _row(i):
            for c in range(O // 16):
                sl = (i, pl.ds(c * 16, 16))
                acc_v[sl] = acc_v[sl] + g1_v[sl]

        pltpu.sync_copy(acc_v, final_hbm.at[pl.ds(base, _RPW)])

    return _sc_combine


def kernel(x, W_out, b_out, W_bid, b_bid):
    bids, idx, pos, bmap, tot, xb = _route(x, W_bid, b_bid)
    pos0, pos1 = pos[:, 0], pos[:, 1]
    xs = _make_sc_dispatch()(xb, pos0, pos1)
    ys = _grouped_mm(bmap.reshape(NBLKMAX), tot.reshape(1), xs, W_out, b_out)
    final = _make_sc_combine()(ys, pos0, pos1)
    return final, idx, bids


# speedup vs baseline: 1.0298x; 1.0170x over previous
import functools

import jax
import jax.numpy as jnp
from jax import lax
from jax.experimental import pallas as pl
from jax.experimental.pallas import tpu as pltpu
from jax.experimental.pallas import tpu_sc as plsc

B = 2048
D = 768
O = 768
E = 16
BLKR = 256
NBLKMAX = (B * 2) // BLKR + E
XSROWS = NBLKMAX * BLKR
HD = D // 2
NEG_INF = float("-inf")

_NC = 2
_NS = 16
_NW = _NC * _NS
_RPW = B // _NW


def _route_kernel(x_ref, w_bid_ref, b_bid_ref,
                  bids_ref, idx_ref, pos_ref, bmap_ref, tot_ref, xb_ref):
    iota = jax.lax.broadcasted_iota(jnp.int32, (B, E), 1)
    x = x_ref[...]
    bids = jax.lax.dot_general(
        x, w_bid_ref[...], (((1,), (1,)), ((), ())),
        preferred_element_type=jnp.float32) + b_bid_ref[...]
    bids_ref[...] = bids
    max0 = jnp.max(bids, axis=1, keepdims=True)
    i0 = jnp.min(jnp.where(bids == max0, iota, E), axis=1, keepdims=True)
    masked = jnp.where(iota == i0, NEG_INF, bids)
    max1 = jnp.max(masked, axis=1, keepdims=True)
    i1 = jnp.min(jnp.where(masked == max1, iota, E), axis=1, keepdims=True)
    idx_ref[...] = jnp.concatenate([i0, i1], axis=1)
    coeff2 = jnp.where((iota == i0) | (iota == i1), 1.0, 0.0)
    s = coeff2
    sh = 1
    while sh < B:
        s = s + jnp.concatenate(
            [jnp.zeros((sh, E), jnp.float32), s[:B - sh, :]], axis=0)
        sh *= 2
    s_excl = s - coeff2
    r0 = jnp.sum(jnp.where(iota == i0, s_excl, 0.0), axis=1, keepdims=True)
    r1 = jnp.sum(jnp.where(iota == i1, s_excl, 0.0), axis=1, keepdims=True)
    totals = s[B - 1:B, :]
    nblk = jnp.floor((totals + (BLKR - 1.0)) * (1.0 / BLKR))
    sb = nblk
    sh = 1
    while sh < E:
        sb = sb + jnp.concatenate(
            [jnp.zeros((1, sh), jnp.float32), sb[:, :E - sh]], axis=1)
        sh *= 2
    start = sb - nblk
    tot = sb[:, E - 1:E]
    off = start * float(BLKR)
    o0 = jnp.sum(jnp.where(iota == i0, off, 0.0), axis=1, keepdims=True)
    o1 = jnp.sum(jnp.where(iota == i1, off, 0.0), axis=1, keepdims=True)
    pos0 = (o0 + r0).astype(jnp.int32)
    pos1 = (o1 + r1).astype(jnp.int32)
    pos_ref[...] = jnp.concatenate([pos0, pos1], axis=1)
    blk_i = jax.lax.broadcasted_iota(jnp.int32, (NBLKMAX, 1), 0
                                     ).astype(jnp.float32)
    blk_i = jnp.minimum(blk_i, tot - 1.0)
    cmp = jnp.where(start <= blk_i, 1.0, 0.0)
    bmap_ref[...] = (jnp.sum(cmp, axis=1, keepdims=True) - 1.0
                     ).astype(jnp.int32)
    tot_ref[...] = tot.astype(jnp.int32)
    xlo = jax.lax.bitcast_convert_type(
        x[:, :HD].astype(jnp.bfloat16).astype(jnp.float32), jnp.uint32)
    xhi = jax.lax.bitcast_convert_type(
        x[:, HD:].astype(jnp.bfloat16).astype(jnp.float32), jnp.uint32)
    xb_ref[...] = jax.lax.bitcast_convert_type(
        xhi | (xlo >> 16), jnp.float32)


def _route(x, W_bid, b_bid):
    return pl.pallas_call(
        _route_kernel,
        in_specs=[
            pl.BlockSpec((B, D), lambda: (0, 0)),
            pl.BlockSpec((E, D), lambda: (0, 0)),
            pl.BlockSpec((1, E), lambda: (0, 0)),
        ],
        out_specs=[
            pl.BlockSpec((B, E), lambda: (0, 0)),
            pl.BlockSpec((B, 2), lambda: (0, 0)),
            pl.BlockSpec((B, 2), lambda: (0, 0)),
            pl.BlockSpec((NBLKMAX, 1), lambda: (0, 0)),
            pl.BlockSpec((1, 1), lambda: (0, 0)),
            pl.BlockSpec((B, HD), lambda: (0, 0)),
        ],
        out_shape=[
            jax.ShapeDtypeStruct((B, E), jnp.float32),
            jax.ShapeDtypeStruct((B, 2), jnp.int32),
            jax.ShapeDtypeStruct((B, 2), jnp.int32),
            jax.ShapeDtypeStruct((NBLKMAX, 1), jnp.int32),
            jax.ShapeDtypeStruct((1, 1), jnp.int32),
            jax.ShapeDtypeStruct((B, HD), jnp.float32),
        ],
    )(x, W_bid, b_bid.reshape(1, E))


@functools.cache
def _sc_mesh():
    return plsc.VectorSubcoreMesh(core_axis_name="c", subcore_axis_name="s",
                                  num_cores=_NC, num_subcores=_NS)


@functools.cache
def _make_sc_dispatch():
    @functools.partial(
        pl.kernel,
        out_type=jax.ShapeDtypeStruct((XSROWS, HD), jnp.float32),
        mesh=_sc_mesh(),
        scratch_types=[
            pltpu.VMEM((_RPW, HD), jnp.float32),
            pltpu.VMEM((_RPW,), jnp.int32),
            pltpu.VMEM((_RPW,), jnp.int32),
            pltpu.SemaphoreType.DMA,
            pltpu.SemaphoreType.DMA,
        ],
    )
    def _sc_dispatch(x_hbm, pos0_hbm, pos1_hbm, xs_hbm,
                     rows_v, p0_v, p1_v, sem0, sem1):
        wid = lax.axis_index("s") * _NC + lax.axis_index("c")
        base = wid * _RPW
        pltpu.sync_copy(x_hbm.at[pl.ds(base, _RPW)], rows_v)
        pltpu.sync_copy(pos0_hbm.at[pl.ds(base, _RPW)], p0_v)
        pltpu.sync_copy(pos1_hbm.at[pl.ds(base, _RPW)], p1_v)
        c0 = pltpu.async_copy(rows_v, xs_hbm.at[p0_v], sem0)
        c1 = pltpu.async_copy(rows_v, xs_hbm.at[p1_v], sem1)
        c0.wait()
        c1.wait()

    return _sc_dispatch


def _mm_body(bmap_ref, tot_ref, xs_ref, w_ref, bo_ref, ys_ref):
    i = pl.program_id(0)

    @pl.when(i < tot_ref[0])
    def _():
        u = jax.lax.bitcast_convert_type(xs_ref[...], jnp.uint32)
        lo = jax.lax.bitcast_convert_type(
            u << 16, jnp.float32).astype(jnp.bfloat16)
        hi = jax.lax.bitcast_convert_type(
            u & jnp.uint32(0xFFFF0000), jnp.float32).astype(jnp.bfloat16)
        w = w_ref[0].astype(jnp.bfloat16)
        dn = (((1,), (0,)), ((), ()))
        acc = (jax.lax.dot_general(lo, w[:HD], dn,
                                   preferred_element_type=jnp.float32)
               + jax.lax.dot_general(hi, w[HD:], dn,
                                     preferred_element_type=jnp.float32))
        ys_ref[...] = 0.5 * acc + 0.5 * bo_ref[0]


def _grouped_mm(bmap, tot, xs, W_out, b_out):
    grid_spec = pltpu.PrefetchScalarGridSpec(
        num_scalar_prefetch=2,
        grid=(NBLKMAX,),
        in_specs=[
            pl.BlockSpec((BLKR, HD),
                         lambda i, bm, tt: (jnp.minimum(i, tt[0] - 1), 0)),
            pl.BlockSpec((1, D, O), lambda i, bm, tt: (bm[i], 0, 0)),
            pl.BlockSpec((1, 1, O), lambda i, bm, tt: (bm[i], 0, 0)),
        ],
        out_specs=pl.BlockSpec((BLKR, O),
                               lambda i, bm, tt: (jnp.minimum(i, tt[0] - 1), 0)),
    )
    return pl.pallas_call(
        _mm_body,
        grid_spec=grid_spec,
        out_shape=jax.ShapeDtypeStruct((XSROWS, O), jnp.float32),
    )(bmap, tot, xs, W_out, b_out.reshape(E, 1, O))


@functools.cache
def _make_sc_combine():
    @functools.partial(
        pl.kernel,
        out_type=jax.ShapeDtypeStruct((B, O), jnp.float32),
        mesh=_sc_mesh(),
        scratch_types=[
            pltpu.VMEM((_RPW, O), jnp.float32),
            pltpu.VMEM((_RPW, O), jnp.float32),
            pltpu.VMEM((_RPW,), jnp.int32),
            pltpu.VMEM((_RPW,), jnp.int32),
            pltpu.SemaphoreType.DMA,
            pltpu.SemaphoreType.DMA,
            pltpu.SemaphoreType.DMA,
        ],
    )
    def _sc_combine(ys_hbm, pos0_hbm, pos1_hbm, final_hbm,
                    acc_v, g1_v, p0_v, p1_v, sem0, sem1, semw):
        wid = lax.axis_index("s") * _NC + lax.axis_index("c")
        base = wid * _RPW
        half = _RPW // 2
        pltpu.sync_copy(pos0_hbm.at[pl.ds(base, _RPW)], p0_v)
        pltpu.sync_copy(pos1_hbm.at[pl.ds(base, _RPW)], p1_v)
        copies = []
        for h in range(2):
            r = pl.ds(h * half, half)
            copies.append((
                pltpu.async_copy(ys_hbm.at[p0_v.at[r]], acc_v.at[r], sem0),
                pltpu.async_copy(ys_hbm.at[p1_v.at[r]], g1_v.at[r], sem1),
            ))
        writes = []
        for h in range(2):
            c0, c1 = copies[h]
            c0.wait()
            c1.wait()

            @plsc.parallel_loop(h * half, (h + 1) * half, 1, unroll=4)
            def _row(i):
                for c in range(O // 16):
                    sl = (i, pl.ds(c * 16, 16))
                    acc_v[sl] = acc_v[sl] + g1_v[sl]

            r = pl.ds(h * half, half)
            writes.append(pltpu.async_copy(
                acc_v.at[r], final_hbm.at[pl.ds(base + h * half, half)],
                semw))
        for wcp in writes:
            wcp.wait()

    return _sc_combine


def kernel(x, W_out, b_out, W_bid, b_bid):
    bids, idx, pos, bmap, tot, xb = _route(x, W_bid, b_bid)
    pos0, pos1 = pos[:, 0], pos[:, 1]
    xs = _make_sc_dispatch()(xb, pos0, pos1)
    ys = _grouped_mm(bmap.reshape(NBLKMAX), tot.reshape(1), xs, W_out, b_out)
    final = _make_sc_combine()(ys, pos0, pos1)
    return final, idx, bids


# speedup vs baseline: 1.0401x; 1.0100x over previous
import functools

import jax
import jax.numpy as jnp
from jax import lax
from jax.experimental import pallas as pl
from jax.experimental.pallas import tpu as pltpu
from jax.experimental.pallas import tpu_sc as plsc

B = 2048
D = 768
O = 768
E = 16
BLKR = 256
NBLKMAX = (B * 2) // BLKR + E
XSROWS = NBLKMAX * BLKR
HD = D // 2
NEG_INF = float("-inf")

_NC = 2
_NS = 16
_NW = _NC * _NS
_RPW = B // _NW


def _route_kernel(x_ref, w_bid_ref, b_bid_ref,
                  bids_ref, idx_ref, pos_ref, bmap_ref, tot_ref, xb_ref):
    iota = jax.lax.broadcasted_iota(jnp.int32, (B, E), 1)
    x = x_ref[...]
    bids = jax.lax.dot_general(
        x, w_bid_ref[...], (((1,), (1,)), ((), ())),
        preferred_element_type=jnp.float32) + b_bid_ref[...]
    bids_ref[...] = bids
    max0 = jnp.max(bids, axis=1, keepdims=True)
    i0 = jnp.min(jnp.where(bids == max0, iota, E), axis=1, keepdims=True)
    masked = jnp.where(iota == i0, NEG_INF, bids)
    max1 = jnp.max(masked, axis=1, keepdims=True)
    i1 = jnp.min(jnp.where(masked == max1, iota, E), axis=1, keepdims=True)
    idx_ref[...] = jnp.concatenate([i0, i1], axis=1)
    coeff2 = jnp.where((iota == i0) | (iota == i1), 1.0, 0.0)
    s = coeff2
    sh = 1
    while sh < B:
        s = s + jnp.concatenate(
            [jnp.zeros((sh, E), jnp.float32), s[:B - sh, :]], axis=0)
        sh *= 2
    s_excl = s - coeff2
    r0 = jnp.sum(jnp.where(iota == i0, s_excl, 0.0), axis=1, keepdims=True)
    r1 = jnp.sum(jnp.where(iota == i1, s_excl, 0.0), axis=1, keepdims=True)
    totals = s[B - 1:B, :]
    nblk = jnp.floor((totals + (BLKR - 1.0)) * (1.0 / BLKR))
    sb = nblk
    sh = 1
    while sh < E:
        sb = sb + jnp.concatenate(
            [jnp.zeros((1, sh), jnp.float32), sb[:, :E - sh]], axis=1)
        sh *= 2
    start = sb - nblk
    tot = sb[:, E - 1:E]
    off = start * float(BLKR)
    o0 = jnp.sum(jnp.where(iota == i0, off, 0.0), axis=1, keepdims=True)
    o1 = jnp.sum(jnp.where(iota == i1, off, 0.0), axis=1, keepdims=True)
    pos0 = (o0 + r0).astype(jnp.int32)
    pos1 = (o1 + r1).astype(jnp.int32)
    pos_ref[...] = jnp.concatenate([pos0, pos1], axis=1)
    blk_i = jax.lax.broadcasted_iota(jnp.int32, (NBLKMAX, 1), 0
                                     ).astype(jnp.float32)
    blk_i = jnp.minimum(blk_i, tot - 1.0)
    cmp = jnp.where(start <= blk_i, 1.0, 0.0)
    bmap_ref[...] = (jnp.sum(cmp, axis=1, keepdims=True) - 1.0
                     ).astype(jnp.int32)
    tot_ref[...] = tot.astype(jnp.int32)
    xlo = jax.lax.bitcast_convert_type(
        x[:, :HD].astype(jnp.bfloat16).astype(jnp.float32), jnp.uint32)
    xhi = jax.lax.bitcast_convert_type(
        x[:, HD:].astype(jnp.bfloat16).astype(jnp.float32), jnp.uint32)
    xb_ref[...] = jax.lax.bitcast_convert_type(
        xhi | (xlo >> 16), jnp.float32)


def _route(x, W_bid, b_bid):
    return pl.pallas_call(
        _route_kernel,
        in_specs=[
            pl.BlockSpec((B, D), lambda: (0, 0)),
            pl.BlockSpec((E, D), lambda: (0, 0)),
            pl.BlockSpec((1, E), lambda: (0, 0)),
        ],
        out_specs=[
            pl.BlockSpec((B, E), lambda: (0, 0)),
            pl.BlockSpec((B, 2), lambda: (0, 0)),
            pl.BlockSpec((B, 2), lambda: (0, 0)),
            pl.BlockSpec((NBLKMAX, 1), lambda: (0, 0)),
            pl.BlockSpec((1, 1), lambda: (0, 0)),
            pl.BlockSpec((B, HD), lambda: (0, 0)),
        ],
        out_shape=[
            jax.ShapeDtypeStruct((B, E), jnp.float32),
            jax.ShapeDtypeStruct((B, 2), jnp.int32),
            jax.ShapeDtypeStruct((B, 2), jnp.int32),
            jax.ShapeDtypeStruct((NBLKMAX, 1), jnp.int32),
            jax.ShapeDtypeStruct((1, 1), jnp.int32),
            jax.ShapeDtypeStruct((B, HD), jnp.float32),
        ],
    )(x, W_bid, b_bid.reshape(1, E))


@functools.cache
def _sc_mesh():
    return plsc.VectorSubcoreMesh(core_axis_name="c", subcore_axis_name="s",
                                  num_cores=_NC, num_subcores=_NS)


@functools.cache
def _make_sc_dispatch():
    @functools.partial(
        pl.kernel,
        out_type=jax.ShapeDtypeStruct((XSROWS, HD), jnp.float32),
        mesh=_sc_mesh(),
        scratch_types=[
            pltpu.VMEM((_RPW, HD), jnp.float32),
            [pltpu.VMEM((_RPW // 2,), jnp.int32) for _ in range(4)],
            pltpu.SemaphoreType.DMA,
            pltpu.SemaphoreType.DMA,
            pltpu.SemaphoreType.DMA,
        ],
    )
    def _sc_dispatch(x_hbm, pos0_hbm, pos1_hbm, xs_hbm,
                     rows_v, idx_vs, sem0, sem1, semr):
        wid = lax.axis_index("s") * _NC + lax.axis_index("c")
        base = wid * _RPW
        half = _RPW // 2
        cr = [pltpu.async_copy(x_hbm.at[pl.ds(base + h * half, half)],
                               rows_v.at[pl.ds(h * half, half)], semr)
              for h in range(2)]
        for h in range(2):
            s = pl.ds(base + h * half, half)
            pltpu.sync_copy(pos0_hbm.at[s], idx_vs[h])
            pltpu.sync_copy(pos1_hbm.at[s], idx_vs[2 + h])
        scat = []
        for h in range(2):
            r = pl.ds(h * half, half)
            cr[h].wait()
            scat.append(pltpu.async_copy(
                rows_v.at[r], xs_hbm.at[idx_vs[h]], sem0))
            scat.append(pltpu.async_copy(
                rows_v.at[r], xs_hbm.at[idx_vs[2 + h]], sem1))
        for c in scat:
            c.wait()

    return _sc_dispatch


def _mm_body(bmap_ref, tot_ref, xs_ref, w_ref, bo_ref, ys_ref):
    i = pl.program_id(0)

    @pl.when(i < tot_ref[0])
    def _():
        u = jax.lax.bitcast_convert_type(xs_ref[...], jnp.uint32)
        lo = jax.lax.bitcast_convert_type(
            u << 16, jnp.float32).astype(jnp.bfloat16)
        hi = jax.lax.bitcast_convert_type(
            u & jnp.uint32(0xFFFF0000), jnp.float32).astype(jnp.bfloat16)
        w = w_ref[0].astype(jnp.bfloat16)
        dn = (((1,), (0,)), ((), ()))
        acc = (jax.lax.dot_general(lo, w[:HD], dn,
                                   preferred_element_type=jnp.float32)
               + jax.lax.dot_general(hi, w[HD:], dn,
                                     preferred_element_type=jnp.float32))
        ys_ref[...] = 0.5 * acc + 0.5 * bo_ref[0]


def _grouped_mm(bmap, tot, xs, W_out, b_out):
    grid_spec = pltpu.PrefetchScalarGridSpec(
        num_scalar_prefetch=2,
        grid=(NBLKMAX,),
        in_specs=[
            pl.BlockSpec((BLKR, HD),
                         lambda i, bm, tt: (jnp.minimum(i, tt[0] - 1), 0)),
            pl.BlockSpec((1, D, O), lambda i, bm, tt: (bm[i], 0, 0)),
            pl.BlockSpec((1, 1, O), lambda i, bm, tt: (bm[i], 0, 0)),
        ],
        out_specs=pl.BlockSpec((BLKR, O),
                               lambda i, bm, tt: (jnp.minimum(i, tt[0] - 1), 0)),
    )
    return pl.pallas_call(
        _mm_body,
        grid_spec=grid_spec,
        out_shape=jax.ShapeDtypeStruct((XSROWS, O), jnp.float32),
    )(bmap, tot, xs, W_out, b_out.reshape(E, 1, O))


@functools.cache
def _make_sc_combine():
    @functools.partial(
        pl.kernel,
        out_type=jax.ShapeDtypeStruct((B, O), jnp.float32),
        mesh=_sc_mesh(),
        scratch_types=[
            pltpu.VMEM((_RPW, O), jnp.float32),
            pltpu.VMEM((_RPW, O), jnp.float32),
            pltpu.VMEM((_RPW,), jnp.int32),
            pltpu.VMEM((_RPW,), jnp.int32),
            pltpu.SemaphoreType.DMA,
            pltpu.SemaphoreType.DMA,
            pltpu.SemaphoreType.DMA,
        ],
    )
    def _sc_combine(ys_hbm, pos0_hbm, pos1_hbm, final_hbm,
                    acc_v, g1_v, p0_v, p1_v, sem0, sem1, semw):
        wid = lax.axis_index("s") * _NC + lax.axis_index("c")
        base = wid * _RPW
        half = _RPW // 2
        pltpu.sync_copy(pos0_hbm.at[pl.ds(base, _RPW)], p0_v)
        pltpu.sync_copy(pos1_hbm.at[pl.ds(base, _RPW)], p1_v)
        copies = []
        for h in range(2):
            r = pl.ds(h * half, half)
            copies.append((
                pltpu.async_copy(ys_hbm.at[p0_v.at[r]], acc_v.at[r], sem0),
                pltpu.async_copy(ys_hbm.at[p1_v.at[r]], g1_v.at[r], sem1),
            ))
        writes = []
        for h in range(2):
            c0, c1 = copies[h]
            c0.wait()
            c1.wait()

            @plsc.parallel_loop(h * half, (h + 1) * half, 1, unroll=4)
            def _row(i):
                for c in range(O // 16):
                    sl = (i, pl.ds(c * 16, 16))
                    acc_v[sl] = acc_v[sl] + g1_v[sl]

            r = pl.ds(h * half, half)
            writes.append(pltpu.async_copy(
                acc_v.at[r], final_hbm.at[pl.ds(base + h * half, half)],
                semw))
        for wcp in writes:
            wcp.wait()

    return _sc_combine


def kernel(x, W_out, b_out, W_bid, b_bid):
    bids, idx, pos, bmap, tot, xb = _route(x, W_bid, b_bid)
    pos0, pos1 = pos[:, 0], pos[:, 1]
    xs = _make_sc_dispatch()(xb, pos0, pos1)
    ys = _grouped_mm(bmap.reshape(NBLKMAX), tot.reshape(1), xs, W_out, b_out)
    final = _make_sc_combine()(ys, pos0, pos1)
    return final, idx, bids
